# baseline (speedup 1.0000x reference)
import functools

import jax
import jax.numpy as jnp
from jax import lax
from jax.experimental import pallas as pl
from jax.experimental.pallas import tpu as pltpu
from jax.experimental.pallas import tpu_sc as plsc

N, E, D, H, DE = 10000, 320000, 128, 128, 16
D_STATE, D_CONV, DT_RANK, D_INNER = 64, 4, 8, 128
NORM = 100.0
f32 = jnp.float32

NC, NS = 2, 16
NW = NC * NS
NSLC = 2
E2 = E // NSLC
EPT = E2 // NW
GCH = 40
GNCH = EPT // GCH
EPC = E2 // NS
SCH = 80
SNCH = EPC // SCH
NPAD = 10240
NPT = NPAD // NS


def _silu(x):
    return x * jax.nn.sigmoid(x)


def _precompute(h, wa, wb):
    TA = 2000

    def body(h_ref, wa_ref, wb_ref, g_ref):
        hh = h_ref[...]
        g_ref[0] = jnp.dot(hh, wa_ref[...], preferred_element_type=f32)
        g_ref[1] = jnp.dot(hh, wb_ref[...], preferred_element_type=f32)

    return pl.pallas_call(
        body,
        grid=(N // TA,),
        in_specs=[
            pl.BlockSpec((TA, D), lambda i: (i, 0)),
            pl.BlockSpec((D, H), lambda i: (0, 0)),
            pl.BlockSpec((D, H), lambda i: (0, 0)),
        ],
        out_specs=pl.BlockSpec((2, TA, H), lambda i: (0, i, 0)),
        out_shape=jax.ShapeDtypeStruct((2, N, H), f32),
        compiler_params=pltpu.CompilerParams(
            dimension_semantics=("arbitrary",)),
    )(h, wa, wb)


def _gather(tab, idx4):
    mesh = plsc.VectorSubcoreMesh(core_axis_name="c", subcore_axis_name="s")
    G2 = 2 * GCH

    @functools.partial(
        pl.kernel,
        mesh=mesh,
        out_type=[
            jax.ShapeDtypeStruct((E2, H), f32),
            jax.ShapeDtypeStruct((E2, H), f32),
        ],
        scratch_types=[
            pltpu.VMEM((GNCH, G2), jnp.int32),
            pltpu.VMEM((G2, H), f32),
            pltpu.VMEM((G2, H), f32),
            pltpu.SemaphoreType.DMA,
        ],
    )
    def k(tab_h, idx_h, sa_h, sb_h, ix_v, a0_v, a1_v, sem):
        c = lax.axis_index("c")
        s = lax.axis_index("s")
        wid = s * NC + c
        base = wid * EPT
        pltpu.sync_copy(idx_h.at[wid], ix_v)

        def start(j, a_v):
            pltpu.async_copy(tab_h.at[ix_v.at[j]], a_v, sem)

        def drain_store(j, a_v):
            off = base + j * GCH
            pltpu.make_async_copy(tab_h.at[ix_v.at[j]], a_v, sem).wait()
            pltpu.sync_copy(a_v.at[pl.ds(0, GCH)], sa_h.at[pl.ds(off, GCH)])
            pltpu.sync_copy(a_v.at[pl.ds(GCH, GCH)],
                            sb_h.at[pl.ds(off, GCH)])

        start(0, a0_v)

        def body(j, carry):
            even = j % 2 == 0

            @pl.when(even)
            def _():
                start(j + 1, a1_v)
                drain_store(j, a0_v)

            @pl.when(jnp.logical_not(even))
            def _():
                start(j + 1, a0_v)
                drain_store(j, a1_v)

            return carry

        lax.fori_loop(0, GNCH - 1, body, 0)
        drain_store(GNCH - 1, a0_v)

    return k(tab, idx4)


def _edge_mlp(sa, sb, ea, w1c, b1, w2, b2, weit, beir):
    EC = 4000
    ne = sa.shape[0]

    def body(sa_ref, sb_ref, ea_ref, w1c_ref, b1_ref, w2_ref, b2_ref,
             weit_ref, beir_ref, o1_ref, o2_ref):
        pre = (sa_ref[...] + sb_ref[...]
               + jnp.dot(ea_ref[...], w1c_ref[...], preferred_element_type=f32)
               + b1_ref[...])
        t = _silu(pre)
        m = _silu(jnp.dot(t, w2_ref[...], preferred_element_type=f32)
                  + b2_ref[...])
        r = jnp.sum(m * weit_ref[...], axis=1, keepdims=True) + beir_ref[...]
        o1_ref[...] = m * (1.0 / NORM)
        o2_ref[...] = m * jax.nn.sigmoid(r)

    full = lambda i: (0, 0)
    return pl.pallas_call(
        body,
        grid=(ne // EC,),
        in_specs=[
            pl.BlockSpec((EC, H), lambda i: (i, 0)),
            pl.BlockSpec((EC, H), lambda i: (i, 0)),
            pl.BlockSpec((EC, DE), lambda i: (i, 0)),
            pl.BlockSpec((DE, H), full),
            pl.BlockSpec((1, H), full),
            pl.BlockSpec((H, H), full),
            pl.BlockSpec((1, H), full),
            pl.BlockSpec((1, H), full),
            pl.BlockSpec((1, 1), full),
        ],
        out_specs=[
            pl.BlockSpec((EC, H), lambda i: (i, 0)),
            pl.BlockSpec((EC, H), lambda i: (i, 0)),
        ],
        out_shape=[
            jax.ShapeDtypeStruct((ne, H), f32),
            jax.ShapeDtypeStruct((ne, H), f32),
        ],
        compiler_params=pltpu.CompilerParams(
            dimension_semantics=("arbitrary",)),
    )(sa, sb, ea, w1c, b1, w2, b2, weit, beir)


def _scatter(o1, o2, row3, col3, zrows):
    mesh = plsc.VectorSubcoreMesh(core_axis_name="c", subcore_axis_name="s")

    @functools.partial(
        pl.kernel,
        mesh=mesh,
        out_type=[
            jax.ShapeDtypeStruct((NPAD, H), f32),
            jax.ShapeDtypeStruct((NPAD, H), f32),
        ],
        scratch_types=[
            pltpu.VMEM((SNCH, SCH), jnp.int32),
            pltpu.VMEM((SCH, H), f32),
            pltpu.VMEM((SCH, H), f32),
            pltpu.VMEM_SHARED((NPAD, H), f32),
            pltpu.SemaphoreType.DMA,
        ],
    )
    def k(o1_h, o2_h, row_h, col_h, z_h, agg_h, mi_h, idx_v, v0_v, v1_v,
          acc, sem):
        c = lax.axis_index("c")
        s = lax.axis_index("s")

        def run(vals_h, idx3_h, out_h):
            pltpu.sync_copy(z_h, acc.at[pl.ds(s * NPT, NPT)])
            pltpu.sync_copy(idx3_h.at[s], idx_v)
            plsc.subcore_barrier()

            def start(j, v_v):
                pltpu.async_copy(
                    vals_h.at[pl.ds(s * EPC + j * SCH, SCH)], v_v, sem)

            def drain_add(j, v_v):
                pltpu.make_async_copy(
                    vals_h.at[pl.ds(s * EPC + j * SCH, SCH)], v_v,
                    sem).wait()
                pltpu.sync_copy(v_v, acc.at[idx_v.at[j]], add=True)

            start(0, v0_v)

            def body(j, carry):
                even = j % 2 == 0

                @pl.when(even)
                def _():
                    start(j + 1, v1_v)
                    drain_add(j, v0_v)

                @pl.when(jnp.logical_not(even))
                def _():
                    start(j + 1, v0_v)
                    drain_add(j, v1_v)

                return carry

            lax.fori_loop(0, SNCH - 1, body, 0)
            drain_add(SNCH - 1, v0_v)

            plsc.subcore_barrier()
            pltpu.sync_copy(acc.at[pl.ds(s * NPT, NPT)],
                            out_h.at[pl.ds(s * NPT, NPT)])

        @pl.when(c == 0)
        def _():
            run(o1_h, row_h, agg_h)

        @pl.when(c == 1)
        def _():
            run(o2_h, col_h, mi_h)

    return k(o1, o2, row3, col3, zrows)


def _node_mamba(h, agg1, agg2, mi1, mi2, wn1a, wn1b, bn1, wn2, bn2, win,
                wcvt, bcv, wxdt, wdt, bdt, wxbc, alogt, dparam, wout):
    T = 400
    CPAD = 8

    def body(h_ref, agg1_ref, agg2_ref, mi1_ref, mi2_ref, wn1a_ref,
             wn1b_ref, bn1_ref, wn2_ref, bn2_ref, win_ref, wcv_ref, bcv_ref,
             wxdt_ref, wdt_ref, bdt_ref, wxbc_ref, alogt_ref, dp_ref,
             wout_ref, out_ref, cc_ref, hs_ref, da3_ref, ou3_ref, hsa_ref):
        i = pl.program_id(0)

        @pl.when(i == 0)
        def _init():
            cc_ref[...] = jnp.zeros((CPAD, D_INNER), f32)
            hs_ref[...] = jnp.zeros((D_STATE, D_INNER), f32)

        hh = h_ref[...]
        agg = agg1_ref[...] + agg2_ref[...]
        mi = mi1_ref[...] + mi2_ref[...]
        t1 = _silu(jnp.dot(hh, wn1a_ref[...], preferred_element_type=f32)
                   + jnp.dot(agg, wn1b_ref[...],
                             preferred_element_type=f32)
                   + bn1_ref[...])
        h2 = hh + jnp.dot(t1, wn2_ref[...], preferred_element_type=f32) \
            + bn2_ref[...]
        t2 = _silu(jnp.dot(mi, wn1a_ref[...],
                           preferred_element_type=f32)
                   + jnp.dot(h2, wn1b_ref[...], preferred_element_type=f32)
                   + bn1_ref[...])
        h3 = jnp.dot(t2, wn2_ref[...], preferred_element_type=f32) \
            + bn2_ref[...]

        xz = jnp.dot(h3, win_ref[...], preferred_element_type=f32)
        x_raw = xz[:, :D_INNER]
        z = xz[:, D_INNER:]

        prev = cc_ref[0:D_CONV - 1, :]
        xpad = jnp.concatenate([prev, x_raw], axis=0)
        acc = jnp.zeros((T, D_INNER), f32) + bcv_ref[...]
        for kk in range(D_CONV):
            acc = acc + xpad[kk:kk + T, :] * wcv_ref[kk:kk + 1, :]
        x = _silu(acc)
        cc_ref[0:D_CONV - 1, :] = x_raw[T - (D_CONV - 1):T, :]

        dt = jax.nn.softplus(
            jnp.dot(jnp.dot(x, wxdt_ref[...], preferred_element_type=f32),
                    wdt_ref[...], preferred_element_type=f32)
            + bdt_ref[...])
        dtx = dt * x
        bc = jnp.dot(x, wxbc_ref[...], preferred_element_type=f32)
        b3 = bc[:, 0:D_STATE][:, :, None]
        c3 = bc[:, D_STATE:][:, :, None]
        atn = -jnp.exp(alogt_ref[...])

        da3_ref[...] = jnp.exp(atn[None, :, :] * dt[:, None, :])
        ou3_ref[...] = b3 * dtx[:, None, :]

        def step(t, hs):
            hs = da3_ref[t] * hs + ou3_ref[t]
            hsa_ref[t] = hs
            return hs

        hs_ref[...] = lax.fori_loop(0, T, step, hs_ref[...])

        ys = jnp.sum(hsa_ref[...] * c3, axis=1)
        y = (ys + dp_ref[...] * x) * _silu(z)
        out_ref[...] = jnp.dot(y, wout_ref[...], preferred_element_type=f32)

    full = lambda i: (0, 0)
    return pl.pallas_call(
        body,
        grid=(N // T,),
        in_specs=[
            pl.BlockSpec((T, D), lambda i: (i, 0)),
            pl.BlockSpec((T, H), lambda i: (i, 0)),
            pl.BlockSpec((T, H), lambda i: (i, 0)),
            pl.BlockSpec((T, H), lambda i: (i, 0)),
            pl.BlockSpec((T, H), lambda i: (i, 0)),
            pl.BlockSpec((D, H), full),
            pl.BlockSpec((D, H), full),
            pl.BlockSpec((1, H), full),
            pl.BlockSpec((H, D), full),
            pl.BlockSpec((1, D), full),
            pl.BlockSpec((D, 2 * D_INNER), full),
            pl.BlockSpec((D_CONV, D_INNER), full),
            pl.BlockSpec((1, D_INNER), full),
            pl.BlockSpec((D_INNER, DT_RANK), full),
            pl.BlockSpec((DT_RANK, D_INNER), full),
            pl.BlockSpec((1, D_INNER), full),
            pl.BlockSpec((D_INNER, 2 * D_STATE), full),
            pl.BlockSpec((D_STATE, D_INNER), full),
            pl.BlockSpec((1, D_INNER), full),
            pl.BlockSpec((D_INNER, H), full),
        ],
        out_specs=pl.BlockSpec((T, H), lambda i: (i, 0)),
        out_shape=jax.ShapeDtypeStruct((N, H), f32),
        scratch_shapes=[
            pltpu.VMEM((CPAD, D_INNER), f32),
            pltpu.VMEM((D_STATE, D_INNER), f32),
            pltpu.VMEM((T, D_STATE, D_INNER), f32),
            pltpu.VMEM((T, D_STATE, D_INNER), f32),
            pltpu.VMEM((T, D_STATE, D_INNER), f32),
        ],
        compiler_params=pltpu.CompilerParams(
            dimension_semantics=("arbitrary",)),
    )(h, agg1, agg2, mi1, mi2, wn1a, wn1b, bn1, wn2, bn2, win, wcvt, bcv,
      wxdt, wdt, bdt, wxbc, alogt, dparam, wout)


def kernel(h, edge_index, edge_attr, We1, be1, We2, be2, Wei, bei, Wn1, bn1,
           Wn2, bn2, Win, Wconv, bconv, Wx, Wdt, bdt, A_log, Dparam, Wout):
    row = edge_index[0]
    col = edge_index[1]

    g = _precompute(h, We1[:D], We1[D:2 * D])
    tab = g.reshape(2 * N, H)
    row4 = row.reshape(NSLC, NW, GNCH, GCH)
    col4 = col.reshape(NSLC, NW, GNCH, GCH)
    idx4 = jnp.concatenate([row4, col4 + N], axis=-1)
    row3 = row.reshape(NSLC, NS, SNCH, SCH)
    col3 = col.reshape(NSLC, NS, SNCH, SCH)
    zrows = jnp.zeros((NPT, H), f32)
    w1c = We1[2 * D:]
    b1r = be1.reshape(1, H)
    b2r = be2.reshape(1, H)
    weit = Wei.reshape(1, H)
    beir = bei.reshape(1, 1)

    o1s, o2s = [], []
    for k in range(NSLC):
        sa, sb = _gather(tab, idx4[k])
        o1, o2 = _edge_mlp(sa, sb, edge_attr[k * E2:(k + 1) * E2],
                           w1c, b1r, We2, b2r, weit, beir)
        o1s.append(o1)
        o2s.append(o2)

    parts = [_scatter(o1s[k], o2s[k], row3[k], col3[k], zrows)
             for k in range(NSLC)]
    (agg1, mi1), (agg2, mi2) = parts

    out = _node_mamba(
        h, agg1[:N], agg2[:N], mi1[:N], mi2[:N], Wn1[:D], Wn1[D:],
        bn1.reshape(1, H), Wn2, bn2.reshape(1, D), Win, Wconv.T,
        bconv.reshape(1, D_INNER), Wx[:, :DT_RANK], Wdt,
        bdt.reshape(1, D_INNER), Wx[:, DT_RANK:], A_log.T,
        Dparam.reshape(1, D_INNER), Wout)
    return out


# speedup vs baseline: 1.0164x; 1.0164x over previous
import functools

import jax
import jax.numpy as jnp
from jax import lax
from jax.experimental import pallas as pl
from jax.experimental.pallas import tpu as pltpu
from jax.experimental.pallas import tpu_sc as plsc

N, E, D, H, DE = 10000, 320000, 128, 128, 16
D_STATE, D_CONV, DT_RANK, D_INNER = 64, 4, 8, 128
NORM = 100.0
f32 = jnp.float32

NC, NS = 2, 16
NW = NC * NS
NSLC = 2
E2 = E // NSLC
EPT = E2 // NW
GCH = 40
GNCH = EPT // GCH
EPC = E2 // NS
SCH = 80
SNCH = EPC // SCH
NPAD = 10240
NPT = NPAD // NS


def _silu(x):
    return x * jax.nn.sigmoid(x)


def _precompute(h, wa, wb):
    TA = 2000

    def body(h_ref, wa_ref, wb_ref, ha_ref, hb_ref):
        hh = h_ref[...]
        ha_ref[...] = jnp.dot(hh, wa_ref[...], preferred_element_type=f32)
        hb_ref[...] = jnp.dot(hh, wb_ref[...], preferred_element_type=f32)

    return pl.pallas_call(
        body,
        grid=(N // TA,),
        in_specs=[
            pl.BlockSpec((TA, D), lambda i: (i, 0)),
            pl.BlockSpec((D, H), lambda i: (0, 0)),
            pl.BlockSpec((D, H), lambda i: (0, 0)),
        ],
        out_specs=[
            pl.BlockSpec((TA, H), lambda i: (i, 0)),
            pl.BlockSpec((TA, H), lambda i: (i, 0)),
        ],
        out_shape=[
            jax.ShapeDtypeStruct((N, H), f32),
            jax.ShapeDtypeStruct((N, H), f32),
        ],
        compiler_params=pltpu.CompilerParams(
            dimension_semantics=("arbitrary",)),
    )(h, wa, wb)


def _gather(ha, hb, row4, col4):
    mesh = plsc.VectorSubcoreMesh(core_axis_name="c", subcore_axis_name="s")

    @functools.partial(
        pl.kernel,
        mesh=mesh,
        out_type=[
            jax.ShapeDtypeStruct((E2, H), f32),
            jax.ShapeDtypeStruct((E2, H), f32),
        ],
        scratch_types=[
            pltpu.VMEM((GNCH, GCH), jnp.int32),
            pltpu.VMEM((GNCH, GCH), jnp.int32),
            pltpu.VMEM((GCH, H), f32),
            pltpu.VMEM((GCH, H), f32),
            pltpu.VMEM((GCH, H), f32),
            pltpu.VMEM((GCH, H), f32),
            pltpu.SemaphoreType.DMA,
            pltpu.SemaphoreType.DMA,
        ],
    )
    def k(ha_h, hb_h, row_h, col_h, sa_h, sb_h, ir_v, ic_v,
          a0_v, a1_v, b0_v, b1_v, sema, semb):
        c = lax.axis_index("c")
        s = lax.axis_index("s")
        wid = s * NC + c
        base = wid * EPT
        pltpu.sync_copy(row_h.at[wid], ir_v)
        pltpu.sync_copy(col_h.at[wid], ic_v)

        def start(j, a_v, b_v):
            pltpu.async_copy(ha_h.at[ir_v.at[j]], a_v, sema)
            pltpu.async_copy(hb_h.at[ic_v.at[j]], b_v, semb)

        def drain_store(j, a_v, b_v):
            off = base + j * GCH
            pltpu.make_async_copy(ha_h.at[ir_v.at[j]], a_v, sema).wait()
            pltpu.make_async_copy(hb_h.at[ic_v.at[j]], b_v, semb).wait()
            pltpu.sync_copy(a_v, sa_h.at[pl.ds(off, GCH)])
            pltpu.sync_copy(b_v, sb_h.at[pl.ds(off, GCH)])

        start(0, a0_v, b0_v)

        def body(j, carry):
            even = j % 2 == 0

            @pl.when(even)
            def _():
                start(j + 1, a1_v, b1_v)
                drain_store(j, a0_v, b0_v)

            @pl.when(jnp.logical_not(even))
            def _():
                start(j + 1, a0_v, b0_v)
                drain_store(j, a1_v, b1_v)

            return carry

        lax.fori_loop(0, GNCH - 1, body, 0)
        drain_store(GNCH - 1, a0_v, b0_v)

    return k(ha, hb, row4, col4)


def _edge_mlp(sa, sb, ea, w1c, b1, w2, b2, weit, beir):
    EC = 4000
    ne = sa.shape[0]

    def body(sa_ref, sb_ref, ea_ref, w1c_ref, b1_ref, w2_ref, b2_ref,
             weit_ref, beir_ref, o1_ref, o2_ref):
        pre = (sa_ref[...] + sb_ref[...]
               + jnp.dot(ea_ref[...], w1c_ref[...], preferred_element_type=f32)
               + b1_ref[...])
        t = _silu(pre)
        m = _silu(jnp.dot(t, w2_ref[...], preferred_element_type=f32)
                  + b2_ref[...])
        r = jnp.sum(m * weit_ref[...], axis=1, keepdims=True) + beir_ref[...]
        o1_ref[...] = m * (1.0 / NORM)
        o2_ref[...] = m * jax.nn.sigmoid(r)

    full = lambda i: (0, 0)
    return pl.pallas_call(
        body,
        grid=(ne // EC,),
        in_specs=[
            pl.BlockSpec((EC, H), lambda i: (i, 0)),
            pl.BlockSpec((EC, H), lambda i: (i, 0)),
            pl.BlockSpec((EC, DE), lambda i: (i, 0)),
            pl.BlockSpec((DE, H), full),
            pl.BlockSpec((1, H), full),
            pl.BlockSpec((H, H), full),
            pl.BlockSpec((1, H), full),
            pl.BlockSpec((1, H), full),
            pl.BlockSpec((1, 1), full),
        ],
        out_specs=[
            pl.BlockSpec((EC, H), lambda i: (i, 0)),
            pl.BlockSpec((EC, H), lambda i: (i, 0)),
        ],
        out_shape=[
            jax.ShapeDtypeStruct((ne, H), f32),
            jax.ShapeDtypeStruct((ne, H), f32),
        ],
        compiler_params=pltpu.CompilerParams(
            dimension_semantics=("arbitrary",)),
    )(sa, sb, ea, w1c, b1, w2, b2, weit, beir)


def _scatter(o1, o2, row3, col3, zrows):
    mesh = plsc.VectorSubcoreMesh(core_axis_name="c", subcore_axis_name="s")

    @functools.partial(
        pl.kernel,
        mesh=mesh,
        out_type=[
            jax.ShapeDtypeStruct((NPAD, H), f32),
            jax.ShapeDtypeStruct((NPAD, H), f32),
        ],
        scratch_types=[
            pltpu.VMEM((SNCH, SCH), jnp.int32),
            pltpu.VMEM((SCH, H), f32),
            pltpu.VMEM((SCH, H), f32),
            pltpu.VMEM_SHARED((NPAD, H), f32),
            pltpu.SemaphoreType.DMA,
        ],
    )
    def k(o1_h, o2_h, row_h, col_h, z_h, agg_h, mi_h, idx_v, v0_v, v1_v,
          acc, sem):
        c = lax.axis_index("c")
        s = lax.axis_index("s")

        def run(vals_h, idx3_h, out_h):
            pltpu.sync_copy(z_h, acc.at[pl.ds(s * NPT, NPT)])
            pltpu.sync_copy(idx3_h.at[s], idx_v)
            plsc.subcore_barrier()

            def start(j, v_v):
                pltpu.async_copy(
                    vals_h.at[pl.ds(s * EPC + j * SCH, SCH)], v_v, sem)

            def drain_add(j, v_v):
                pltpu.make_async_copy(
                    vals_h.at[pl.ds(s * EPC + j * SCH, SCH)], v_v,
                    sem).wait()
                pltpu.sync_copy(v_v, acc.at[idx_v.at[j]], add=True)

            start(0, v0_v)

            def body(j, carry):
                even = j % 2 == 0

                @pl.when(even)
                def _():
                    start(j + 1, v1_v)
                    drain_add(j, v0_v)

                @pl.when(jnp.logical_not(even))
                def _():
                    start(j + 1, v0_v)
                    drain_add(j, v1_v)

                return carry

            lax.fori_loop(0, SNCH - 1, body, 0)
            drain_add(SNCH - 1, v0_v)

            plsc.subcore_barrier()
            pltpu.sync_copy(acc.at[pl.ds(s * NPT, NPT)],
                            out_h.at[pl.ds(s * NPT, NPT)])

        @pl.when(c == 0)
        def _():
            run(o1_h, row_h, agg_h)

        @pl.when(c == 1)
        def _():
            run(o2_h, col_h, mi_h)

    return k(o1, o2, row3, col3, zrows)


def _node_mamba(h, agg1, agg2, mi1, mi2, wn1a, wn1b, bn1, wn2, bn2, win,
                wcvt, bcv, wxdt, wdt, bdt, wxbc, alogt, dparam, wout):
    T = 400
    CPAD = 8

    def body(h_ref, agg1_ref, agg2_ref, mi1_ref, mi2_ref, wn1a_ref,
             wn1b_ref, bn1_ref, wn2_ref, bn2_ref, win_ref, wcv_ref, bcv_ref,
             wxdt_ref, wdt_ref, bdt_ref, wxbc_ref, alogt_ref, dp_ref,
             wout_ref, out_ref, cc_ref, hs_ref, da3_ref, ou3_ref, hsa_ref):
        i = pl.program_id(0)

        @pl.when(i == 0)
        def _init():
            cc_ref[...] = jnp.zeros((CPAD, D_INNER), f32)
            hs_ref[...] = jnp.zeros((D_STATE, D_INNER), f32)

        hh = h_ref[...]
        agg = agg1_ref[...] + agg2_ref[...]
        mi = mi1_ref[...] + mi2_ref[...]
        t1 = _silu(jnp.dot(hh, wn1a_ref[...], preferred_element_type=f32)
                   + jnp.dot(agg, wn1b_ref[...],
                             preferred_element_type=f32)
                   + bn1_ref[...])
        h2 = hh + jnp.dot(t1, wn2_ref[...], preferred_element_type=f32) \
            + bn2_ref[...]
        t2 = _silu(jnp.dot(mi, wn1a_ref[...],
                           preferred_element_type=f32)
                   + jnp.dot(h2, wn1b_ref[...], preferred_element_type=f32)
                   + bn1_ref[...])
        h3 = jnp.dot(t2, wn2_ref[...], preferred_element_type=f32) \
            + bn2_ref[...]

        xz = jnp.dot(h3, win_ref[...], preferred_element_type=f32)
        x_raw = xz[:, :D_INNER]
        z = xz[:, D_INNER:]

        prev = cc_ref[0:D_CONV - 1, :]
        xpad = jnp.concatenate([prev, x_raw], axis=0)
        acc = jnp.zeros((T, D_INNER), f32) + bcv_ref[...]
        for kk in range(D_CONV):
            acc = acc + xpad[kk:kk + T, :] * wcv_ref[kk:kk + 1, :]
        x = _silu(acc)
        cc_ref[0:D_CONV - 1, :] = x_raw[T - (D_CONV - 1):T, :]

        dt = jax.nn.softplus(
            jnp.dot(jnp.dot(x, wxdt_ref[...], preferred_element_type=f32),
                    wdt_ref[...], preferred_element_type=f32)
            + bdt_ref[...])
        dtx = dt * x
        bc = jnp.dot(x, wxbc_ref[...], preferred_element_type=f32)
        b3 = bc[:, 0:D_STATE][:, :, None]
        c3 = bc[:, D_STATE:][:, :, None]
        atn = -jnp.exp(alogt_ref[...])

        da3_ref[...] = jnp.exp(atn[None, :, :] * dt[:, None, :])
        ou3_ref[...] = b3 * dtx[:, None, :]

        def step(t, hs):
            hs = da3_ref[t] * hs + ou3_ref[t]
            hsa_ref[t] = hs
            return hs

        hs_ref[...] = lax.fori_loop(0, T, step, hs_ref[...])

        ys = jnp.sum(hsa_ref[...] * c3, axis=1)
        y = (ys + dp_ref[...] * x) * _silu(z)
        out_ref[...] = jnp.dot(y, wout_ref[...], preferred_element_type=f32)

    full = lambda i: (0, 0)
    return pl.pallas_call(
        body,
        grid=(N // T,),
        in_specs=[
            pl.BlockSpec((T, D), lambda i: (i, 0)),
            pl.BlockSpec((T, H), lambda i: (i, 0)),
            pl.BlockSpec((T, H), lambda i: (i, 0)),
            pl.BlockSpec((T, H), lambda i: (i, 0)),
            pl.BlockSpec((T, H), lambda i: (i, 0)),
            pl.BlockSpec((D, H), full),
            pl.BlockSpec((D, H), full),
            pl.BlockSpec((1, H), full),
            pl.BlockSpec((H, D), full),
            pl.BlockSpec((1, D), full),
            pl.BlockSpec((D, 2 * D_INNER), full),
            pl.BlockSpec((D_CONV, D_INNER), full),
            pl.BlockSpec((1, D_INNER), full),
            pl.BlockSpec((D_INNER, DT_RANK), full),
            pl.BlockSpec((DT_RANK, D_INNER), full),
            pl.BlockSpec((1, D_INNER), full),
            pl.BlockSpec((D_INNER, 2 * D_STATE), full),
            pl.BlockSpec((D_STATE, D_INNER), full),
            pl.BlockSpec((1, D_INNER), full),
            pl.BlockSpec((D_INNER, H), full),
        ],
        out_specs=pl.BlockSpec((T, H), lambda i: (i, 0)),
        out_shape=jax.ShapeDtypeStruct((N, H), f32),
        scratch_shapes=[
            pltpu.VMEM((CPAD, D_INNER), f32),
            pltpu.VMEM((D_STATE, D_INNER), f32),
            pltpu.VMEM((T, D_STATE, D_INNER), f32),
            pltpu.VMEM((T, D_STATE, D_INNER), f32),
            pltpu.VMEM((T, D_STATE, D_INNER), f32),
        ],
        compiler_params=pltpu.CompilerParams(
            dimension_semantics=("arbitrary",)),
    )(h, agg1, agg2, mi1, mi2, wn1a, wn1b, bn1, wn2, bn2, win, wcvt, bcv,
      wxdt, wdt, bdt, wxbc, alogt, dparam, wout)


def kernel(h, edge_index, edge_attr, We1, be1, We2, be2, Wei, bei, Wn1, bn1,
           Wn2, bn2, Win, Wconv, bconv, Wx, Wdt, bdt, A_log, Dparam, Wout):
    row = edge_index[0]
    col = edge_index[1]

    ha, hb = _precompute(h, We1[:D], We1[D:2 * D])
    row4 = row.reshape(NSLC, NW, GNCH, GCH)
    col4 = col.reshape(NSLC, NW, GNCH, GCH)
    row3 = row.reshape(NSLC, NS, SNCH, SCH)
    col3 = col.reshape(NSLC, NS, SNCH, SCH)
    zrows = jnp.zeros((NPT, H), f32)
    w1c = We1[2 * D:]
    b1r = be1.reshape(1, H)
    b2r = be2.reshape(1, H)
    weit = Wei.reshape(1, H)
    beir = bei.reshape(1, 1)

    o1s, o2s = [], []
    for k in range(NSLC):
        sa, sb = _gather(ha, hb, row4[k], col4[k])
        o1, o2 = _edge_mlp(sa, sb, edge_attr[k * E2:(k + 1) * E2],
                           w1c, b1r, We2, b2r, weit, beir)
        o1s.append(o1)
        o2s.append(o2)

    parts = [_scatter(o1s[k], o2s[k], row3[k], col3[k], zrows)
             for k in range(NSLC)]
    (agg1, mi1), (agg2, mi2) = parts

    out = _node_mamba(
        h, agg1[:N], agg2[:N], mi1[:N], mi2[:N], Wn1[:D], Wn1[D:],
        bn1.reshape(1, H), Wn2, bn2.reshape(1, D), Win, Wconv.T,
        bconv.reshape(1, D_INNER), Wx[:, :DT_RANK], Wdt,
        bdt.reshape(1, D_INNER), Wx[:, DT_RANK:], A_log.T,
        Dparam.reshape(1, D_INNER), Wout)
    return out
